# baseline (speedup 1.0000x reference)
import functools

import jax
import jax.numpy as jnp
from jax import lax
from jax.experimental import pallas as pl
from jax.experimental.pallas import tpu as pltpu
from jax.experimental.pallas import tpu_sc as plsc

DAY_LEN = 288
D_MODEL = 64

NC = 2
NS = 16
L = 16
NW = NC * NS

NB = 8
G1 = 104


def _sc_lookup(n_rows: int, n_cols: int):
  rows_per_w = n_rows // NW
  t_total = rows_per_w // NB
  t2_total = t_total // 2
  g2 = n_cols - G1
  n_grp = n_cols // L
  mesh = plsc.VectorSubcoreMesh(
      core_axis_name="c", subcore_axis_name="s", num_cores=NC,
      num_subcores=NS)

  @functools.partial(
      pl.kernel,
      mesh=mesh,
      compiler_params=pltpu.CompilerParams(use_tc_tiling_on_sc=False),
      out_type=jax.ShapeDtypeStruct((n_rows, n_cols, D_MODEL), jnp.float32),
      scratch_types=(
          [
              pltpu.VMEM((2, NB, n_cols), jnp.float32),
              pltpu.VMEM((NB, n_cols), jnp.int32),
              pltpu.VMEM((NB, n_cols, D_MODEL), jnp.float32),
          ]
          + [pltpu.SemaphoreType.DMA] * (2 + NB + NB)
      ),
  )
  def k(x_hbm, table_hbm, out_hbm, x_v, idx_v, rows_v, *sems):
    sem_x = sems[0:2]
    sem_g = sems[2:2 + NB]
    sem_o = sems[2 + NB:2 + 2 * NB]
    wid = lax.axis_index("s") * NC + lax.axis_index("c")
    row0 = wid * rows_per_w

    def x_copy(t, xb):
      return pltpu.make_async_copy(
          x_hbm.at[pl.ds(row0 + t * NB, NB), :], x_v.at[xb], sem_x[xb])

    def gather(b, part):
      lo, ln = (0, G1) if part == 0 else (G1, g2)
      return pltpu.make_async_copy(
          table_hbm.at[idx_v.at[b, pl.ds(lo, ln)]],
          rows_v.at[b, pl.ds(lo, ln), :], sem_g[b])

    def out_copy(t, b):
      return pltpu.make_async_copy(
          rows_v.at[b], out_hbm.at[row0 + t * NB + b], sem_o[b])

    def ring_iter(t, xb):
      x_copy(t, xb).wait()

      @pl.when(t + 1 < t_total)
      def _():
        x_copy(t + 1, 1 - xb).start()

      for b in range(NB):
        offs = [j * L for j in range(n_grp)]
        if n_cols % L:
          offs.append(n_cols - L)
        for off in offs:
          xv = x_v[xb, b, pl.ds(off, L)]
          iv = jnp.clip((xv * float(DAY_LEN)).astype(jnp.int32), 0,
                        DAY_LEN - 1)
          idx_v[b, pl.ds(off, L)] = iv

        @pl.when(t > 0)
        def _(b=b):
          out_copy(t - 1, b).wait()

        gather(b, 0).start()
        gather(b, 1).start()
      for b in range(NB):
        gather(b, 0).wait()
        gather(b, 1).wait()
        out_copy(t, b).start()

    x_copy(0, 0).start()

    def outer(t2, carry):
      t = t2 * 2
      ring_iter(t, 0)
      ring_iter(t + 1, 1)
      return carry

    lax.fori_loop(0, t2_total, outer, 0)

    for b in range(NB):
      out_copy(t_total - 1, b).wait()

  return k


def kernel(x_time_norm, day_emb):
  n_rows, n_cols = x_time_norm.shape
  return _sc_lookup(n_rows, n_cols)(x_time_norm, day_emb)


# speedup vs baseline: 2.1328x; 2.1328x over previous
import functools

import jax
import jax.numpy as jnp
from jax import lax
from jax.experimental import pallas as pl
from jax.experimental.pallas import tpu as pltpu
from jax.experimental.pallas import tpu_sc as plsc

DAY_LEN = 288
D_MODEL = 64

NC = 2
NS = 16
L = 16
NW = NC * NS

DT = D_MODEL // 8


def _sc_lookup(n_b: int, n_t: int):
  b_per_w = n_b // NW
  bt_per_w = b_per_w // 128
  n_bt = n_b // 128
  mesh = plsc.VectorSubcoreMesh(
      core_axis_name="c", subcore_axis_name="s", num_cores=NC,
      num_subcores=NS)

  @functools.partial(
      pl.kernel,
      mesh=mesh,
      compiler_params=pltpu.CompilerParams(
          use_tc_tiling_on_sc=False, needs_layout_passes=False),
      out_type=jax.ShapeDtypeStruct((n_t, DT, n_bt, 8, 128), jnp.float32),
      scratch_types=[
          pltpu.VMEM((DAY_LEN * D_MODEL,), jnp.float32),
          pltpu.VMEM((2, b_per_w), jnp.float32),
          pltpu.VMEM((2, DT, bt_per_w, 8, 128), jnp.float32),
          pltpu.SemaphoreType.DMA,
          pltpu.SemaphoreType.DMA,
          pltpu.SemaphoreType.DMA,
          pltpu.SemaphoreType.DMA,
      ],
  )
  def k(xt_hbm, table_hbm, out_hbm, table_v, x_v, stage_v, sx0, sx1, so0, so1):
    sem_x = (sx0, sx1)
    sem_o = (so0, so1)
    wid = lax.axis_index("s") * NC + lax.axis_index("c")
    b0 = wid * b_per_w
    bt0 = wid * bt_per_w

    pltpu.sync_copy(table_hbm, table_v)

    def x_copy(t, xb):
      return pltpu.make_async_copy(
          xt_hbm.at[t, pl.ds(b0, b_per_w)], x_v.at[xb], sem_x[xb])

    def out_copy(t, s):
      return pltpu.make_async_copy(
          stage_v.at[s],
          out_hbm.at[t, :, pl.ds(bt0, bt_per_w), :, :], sem_o[s])

    def half(t, s):
      x_copy(t, s).wait()

      @pl.when(t + 1 < n_t)
      def _():
        x_copy(t + 1, 1 - s).start()

      @pl.when(t >= 2)
      def _():
        out_copy(t - 2, s).wait()

      def bt_body(btw, carry):
        boff = btw * 128
        for gr in range(8):
          xv = x_v[s, pl.ds(boff + gr * L, L)]
          iv = jnp.clip((xv * float(DAY_LEN)).astype(jnp.int32), 0,
                        DAY_LEN - 1)
          for d in range(D_MODEL):
            v = plsc.load_gather(
                table_v.at[pl.ds(d * DAY_LEN, DAY_LEN)], [iv])
            stage_v[s, d // 8, btw, d % 8, pl.ds(gr * L, L)] = v
        return carry

      lax.fori_loop(0, bt_per_w, bt_body, 0)
      out_copy(t, s).start()

    x_copy(0, 0).start()

    def outer(t2, carry):
      half(t2 * 2, 0)
      half(t2 * 2 + 1, 1)
      return carry

    lax.fori_loop(0, n_t // 2, outer, 0)
    out_copy(n_t - 2, 0).wait()
    out_copy(n_t - 1, 1).wait()

  return k


def kernel(x_time_norm, day_emb):
  n_b, n_t = x_time_norm.shape
  xt = jnp.transpose(x_time_norm)
  table_t = jnp.transpose(day_emb).reshape(-1)
  out5 = _sc_lookup(n_b, n_t)(xt, table_t)
  out = jnp.transpose(out5, (2, 4, 0, 1, 3))
  return out.reshape(n_b, n_t, D_MODEL)
